# Initial kernel scaffold; baseline (speedup 1.0000x reference)
import jax
import jax.numpy as jnp
from jax.experimental import pallas as pl


def kernel(input, edge_index, edge_weight, W, b):
    raise NotImplementedError("write your pallas kernel here")


# speedup vs baseline: 2.6150x; 2.6150x over previous
import functools

import jax
import jax.numpy as jnp
from jax import lax
from jax.experimental import pallas as pl
from jax.experimental.pallas import tpu as pltpu
from jax.experimental.pallas import tpu_sc as plsc

NC = 2
NS = 16
L = 16

CHUNK = 128
COLS = 128


def _linear_tc(x, W, b):
    N, K = x.shape
    DOUT = W.shape[0]
    BM = 400

    def mm(x_ref, w_ref, b_ref, o_ref):
        o_ref[...] = lax.dot_general(
            x_ref[...], w_ref[...],
            dimension_numbers=(((1,), (1,)), ((), ())),
            preferred_element_type=jnp.float32,
        ) + b_ref[...]

    return pl.pallas_call(
        mm,
        grid=(N // BM,),
        in_specs=[
            pl.BlockSpec((BM, K), lambda i: (i, 0)),
            pl.BlockSpec((DOUT, K), lambda i: (0, 0)),
            pl.BlockSpec((1, DOUT), lambda i: (0, 0)),
        ],
        out_specs=pl.BlockSpec((BM, DOUT), lambda i: (i, 0)),
        out_shape=jax.ShapeDtypeStruct((N, DOUT), jnp.float32),
    )(x, W, b[None, :])


def _spmm_sc(sup2, dst, src, w, n_nodes, n_chunks):
    rows_per_tile = n_nodes // NS
    edges_per_tile = n_chunks * CHUNK
    OB = 125

    mesh = plsc.VectorSubcoreMesh(core_axis_name="c", subcore_axis_name="s")

    @functools.partial(
        pl.kernel,
        out_type=jax.ShapeDtypeStruct((n_nodes, NC, COLS), jnp.float32),
        mesh=mesh,
        scratch_types=[
            pltpu.VMEM_SHARED((n_nodes, COLS), jnp.float32),
            pltpu.VMEM((n_chunks, CHUNK), jnp.int32),
            pltpu.VMEM((n_chunks, CHUNK), jnp.int32),
            pltpu.VMEM((edges_per_tile,), jnp.float32),
            pltpu.VMEM((CHUNK, COLS), jnp.float32),
            pltpu.SemaphoreType.DMA,
        ],
    )
    def k(sup_hbm, dst_hbm, src_hbm, w_hbm, out_hbm,
          acc, src_v, dst_v, w_v, rows_v, sem):
        c = lax.axis_index("c")
        s = lax.axis_index("s")
        ebase = s * edges_per_tile
        rbase = s * rows_per_tile

        def zrow(i, _):
            for g in range(COLS // L):
                rows_v[i, pl.ds(g * L, L)] = jnp.zeros((L,), jnp.float32)
            return 0
        lax.fori_loop(0, OB, zrow, 0)
        for j in range(rows_per_tile // OB):
            pltpu.sync_copy(rows_v.at[pl.ds(0, OB)], acc.at[pl.ds(rbase + j * OB, OB)])

        pltpu.sync_copy(src_hbm.at[pl.ds(s * n_chunks, n_chunks)], src_v)
        pltpu.sync_copy(dst_hbm.at[pl.ds(s * n_chunks, n_chunks)], dst_v)
        pltpu.sync_copy(w_hbm.at[pl.ds(ebase, edges_per_tile)], w_v)

        def gi(i, _):
            for g in range(CHUNK // L):
                s16 = src_v[i, pl.ds(g * L, L)]
                src_v[i, pl.ds(g * L, L)] = s16 * 2 + c
            return 0
        lax.fori_loop(0, n_chunks, gi, 0)

        plsc.subcore_barrier()

        def chunk(i, _):
            pltpu.async_copy(sup_hbm.at[src_v.at[i]], rows_v, sem).wait()

            def scale(e16, _):
                w16 = w_v[pl.ds(i * CHUNK + e16 * L, L)]
                for j in range(L):
                    wt = w16[j]
                    e = e16 * L + j
                    for g in range(COLS // L):
                        rows_v[e, pl.ds(g * L, L)] = rows_v[e, pl.ds(g * L, L)] * wt
                return 0
            lax.fori_loop(0, CHUNK // L, scale, 0)

            pltpu.sync_copy(rows_v, acc.at[dst_v.at[i]], add=True)
            return 0
        lax.fori_loop(0, n_chunks, chunk, 0)

        plsc.subcore_barrier()

        for j in range(rows_per_tile // OB):
            r0 = rbase + j * OB
            pltpu.sync_copy(acc.at[pl.ds(r0, OB)], rows_v.at[pl.ds(0, OB)])
            pltpu.sync_copy(rows_v.at[pl.ds(0, OB)], out_hbm.at[pl.ds(r0, OB), c])

    return k(sup2, dst, src, w)


def kernel(input, edge_index, edge_weight, W, b):
    n_nodes, d_in = input.shape
    n_edges = edge_weight.shape[0]

    support = _linear_tc(input, W, b)
    sup2 = support.reshape(n_nodes * NC, COLS)

    grain = NS * CHUNK * 8
    ep = ((n_edges + grain - 1) // grain) * grain
    pad = ep - n_edges
    dst = jnp.pad(edge_index[0], (0, pad)).reshape(ep // CHUNK, CHUNK)
    src = jnp.pad(edge_index[1], (0, pad)).reshape(ep // CHUNK, CHUNK)
    w = jnp.pad(edge_weight, (0, pad))

    out3 = _spmm_sc(sup2, dst, src, w, n_nodes, ep // (NS * CHUNK))
    return out3.reshape(n_nodes, NC * COLS)


# speedup vs baseline: 3.1328x; 1.1980x over previous
import functools

import jax
import jax.numpy as jnp
from jax import lax
from jax.experimental import pallas as pl
from jax.experimental.pallas import tpu as pltpu
from jax.experimental.pallas import tpu_sc as plsc

NC = 2
NS = 16
L = 16

CHUNK = 128
COLS = 128


def _linear_tc(x, W, b):
    N, K = x.shape
    DOUT = W.shape[0]
    BM = 400

    def mm(x_ref, w_ref, b_ref, o_ref):
        o_ref[...] = lax.dot_general(
            x_ref[...], w_ref[...],
            dimension_numbers=(((1,), (1,)), ((), ())),
            preferred_element_type=jnp.float32,
        ) + b_ref[...]

    return pl.pallas_call(
        mm,
        grid=(N // BM,),
        in_specs=[
            pl.BlockSpec((BM, K), lambda i: (i, 0)),
            pl.BlockSpec((DOUT, K), lambda i: (0, 0)),
            pl.BlockSpec((1, DOUT), lambda i: (0, 0)),
        ],
        out_specs=pl.BlockSpec((BM, DOUT), lambda i: (i, 0)),
        out_shape=jax.ShapeDtypeStruct((N, DOUT), jnp.float32),
    )(x, W, b[None, :])


def _spmm_sc(sup2, dst, src, w, n_nodes, n_chunks):
    rows_per_tile = n_nodes // NS
    edges_per_tile = n_chunks * CHUNK
    OB = 125
    G = 16
    n_groups = n_chunks // G

    mesh = plsc.VectorSubcoreMesh(core_axis_name="c", subcore_axis_name="s")

    @functools.partial(
        pl.kernel,
        out_type=jax.ShapeDtypeStruct((n_nodes, NC, COLS), jnp.float32),
        mesh=mesh,
        scratch_types=[
            pltpu.VMEM_SHARED((n_nodes, COLS), jnp.float32),
            pltpu.VMEM((G, CHUNK), jnp.int32),
            pltpu.VMEM((G, CHUNK), jnp.int32),
            pltpu.VMEM((G * CHUNK,), jnp.float32),
            pltpu.VMEM((CHUNK, COLS), jnp.float32),
            pltpu.VMEM((CHUNK, COLS), jnp.float32),
            pltpu.SemaphoreType.DMA,
            pltpu.SemaphoreType.DMA,
        ],
    )
    def k(sup_hbm, dst_hbm, src_hbm, w_hbm, out_hbm,
          acc, src_v, dst_v, w_v, rows0_v, rows1_v, sem0, sem1):
        c = lax.axis_index("c")
        s = lax.axis_index("s")
        rbase = s * rows_per_tile

        def zrow(i, _):
            for g in range(COLS // L):
                rows0_v[i, pl.ds(g * L, L)] = jnp.zeros((L,), jnp.float32)
            return 0
        lax.fori_loop(0, OB, zrow, 0)
        for j in range(rows_per_tile // OB):
            pltpu.sync_copy(rows0_v.at[pl.ds(0, OB)], acc.at[pl.ds(rbase + j * OB, OB)])

        plsc.subcore_barrier()

        def scale(rows_v, wbase):
            def body(e16, _):
                w16 = w_v[pl.ds(wbase + e16 * L, L)]
                for j in range(L):
                    wt = w16[j]
                    e = e16 * L + j
                    for g in range(COLS // L):
                        rows_v[e, pl.ds(g * L, L)] = rows_v[e, pl.ds(g * L, L)] * wt
                return 0
            lax.fori_loop(0, CHUNK // L, body, 0)

        def group(gidx, _):
            cbase = s * n_chunks + gidx * G
            ebase = (s * n_chunks + gidx * G) * CHUNK

            pltpu.sync_copy(src_hbm.at[pl.ds(cbase, G)], src_v)
            pltpu.sync_copy(dst_hbm.at[pl.ds(cbase, G)], dst_v)
            pltpu.sync_copy(w_hbm.at[pl.ds(ebase, G * CHUNK)], w_v)

            def gi(i, _):
                for g in range(CHUNK // L):
                    s16 = src_v[i, pl.ds(g * L, L)]
                    src_v[i, pl.ds(g * L, L)] = s16 * 2 + c
                return 0
            lax.fori_loop(0, G, gi, 0)

            pltpu.async_copy(sup_hbm.at[src_v.at[0]], rows0_v, sem0)

            def pair(j, _):
                i0 = 2 * j
                i1 = 2 * j + 1
                pltpu.make_async_copy(sup_hbm.at[src_v.at[i0]], rows0_v, sem0).wait()
                pltpu.async_copy(sup_hbm.at[src_v.at[i1]], rows1_v, sem1)
                scale(rows0_v, i0 * CHUNK)
                pltpu.sync_copy(rows0_v, acc.at[dst_v.at[i0]], add=True)

                pltpu.make_async_copy(sup_hbm.at[src_v.at[i1]], rows1_v, sem1).wait()

                @pl.when(j < G // 2 - 1)
                def _():
                    pltpu.async_copy(sup_hbm.at[src_v.at[i1 + 1]], rows0_v, sem0)

                scale(rows1_v, i1 * CHUNK)
                pltpu.sync_copy(rows1_v, acc.at[dst_v.at[i1]], add=True)
                return 0
            lax.fori_loop(0, G // 2, pair, 0)
            return 0
        lax.fori_loop(0, n_groups, group, 0)

        plsc.subcore_barrier()

        for j in range(rows_per_tile // OB):
            r0 = rbase + j * OB
            pltpu.sync_copy(acc.at[pl.ds(r0, OB)], rows0_v.at[pl.ds(0, OB)])
            pltpu.sync_copy(rows0_v.at[pl.ds(0, OB)], out_hbm.at[pl.ds(r0, OB), c])

    return k(sup2, dst, src, w)


def kernel(input, edge_index, edge_weight, W, b):
    n_nodes, d_in = input.shape
    n_edges = edge_weight.shape[0]

    support = _linear_tc(input, W, b)
    sup2 = support.reshape(n_nodes * NC, COLS)

    grain = NS * CHUNK * 8
    ep = ((n_edges + grain - 1) // grain) * grain
    pad = ep - n_edges
    dst = jnp.pad(edge_index[0], (0, pad)).reshape(ep // CHUNK, CHUNK)
    src = jnp.pad(edge_index[1], (0, pad)).reshape(ep // CHUNK, CHUNK)
    w = jnp.pad(edge_weight, (0, pad))

    out3 = _spmm_sc(sup2, dst, src, w, n_nodes, ep // (NS * CHUNK))
    return out3.reshape(n_nodes, NC * COLS)


# speedup vs baseline: 3.1513x; 1.0059x over previous
import functools

import jax
import jax.numpy as jnp
from jax import lax
from jax.experimental import pallas as pl
from jax.experimental.pallas import tpu as pltpu
from jax.experimental.pallas import tpu_sc as plsc

NC = 2
NS = 16
L = 16

CHUNK = 64
NBUF = 4
GRP = 32
COLS = 128


def _linear_tc(x, W, b):
    N, K = x.shape
    DOUT = W.shape[0]
    BM = 400

    def mm(x_ref, w_ref, b_ref, o_ref):
        o_ref[...] = lax.dot_general(
            x_ref[...], w_ref[...],
            dimension_numbers=(((1,), (1,)), ((), ())),
            preferred_element_type=jnp.float32,
        ) + b_ref[...]

    return pl.pallas_call(
        mm,
        grid=(N // BM,),
        in_specs=[
            pl.BlockSpec((BM, K), lambda i: (i, 0)),
            pl.BlockSpec((DOUT, K), lambda i: (0, 0)),
            pl.BlockSpec((1, DOUT), lambda i: (0, 0)),
        ],
        out_specs=pl.BlockSpec((BM, DOUT), lambda i: (i, 0)),
        out_shape=jax.ShapeDtypeStruct((N, DOUT), jnp.float32),
    )(x, W, b[None, :])


def _spmm_sc(sup2, dst, src, w, n_nodes, n_chunks):
    rows_per_tile = n_nodes // NS
    edges_per_tile = n_chunks * CHUNK
    OB = 25
    n_groups = n_chunks // GRP

    mesh = plsc.VectorSubcoreMesh(core_axis_name="c", subcore_axis_name="s")

    row_bufs = [pltpu.VMEM((CHUNK, COLS), jnp.float32) for _ in range(NBUF)]
    gsems = [pltpu.SemaphoreType.DMA for _ in range(NBUF)]
    ssems = [pltpu.SemaphoreType.DMA for _ in range(NBUF)]

    @functools.partial(
        pl.kernel,
        out_type=jax.ShapeDtypeStruct((n_nodes, NC, COLS), jnp.float32),
        mesh=mesh,
        scratch_types=[
            pltpu.VMEM_SHARED((n_nodes, COLS), jnp.float32),
            pltpu.VMEM((GRP, CHUNK), jnp.int32),
            pltpu.VMEM((GRP, CHUNK), jnp.int32),
            pltpu.VMEM((GRP * CHUNK,), jnp.float32),
        ] + row_bufs + gsems + ssems,
    )
    def k(sup_hbm, dst_hbm, src_hbm, w_hbm, out_hbm,
          acc, src_v, dst_v, w_v, *bufs_and_sems):
        bufs = bufs_and_sems[:NBUF]
        gsem = bufs_and_sems[NBUF:2 * NBUF]
        ssem = bufs_and_sems[2 * NBUF:3 * NBUF]
        c = lax.axis_index("c")
        s = lax.axis_index("s")
        rbase = s * rows_per_tile

        def zrow(i, _):
            for g in range(COLS // L):
                bufs[0][i, pl.ds(g * L, L)] = jnp.zeros((L,), jnp.float32)
            return 0
        lax.fori_loop(0, OB, zrow, 0)
        def zacc(j, _):
            pltpu.sync_copy(bufs[0].at[pl.ds(0, OB)], acc.at[pl.ds(rbase + j * OB, OB)])
            return 0
        lax.fori_loop(0, rows_per_tile // OB, zacc, 0)

        plsc.subcore_barrier()

        def scale(rows_v, wbase):
            def body(e16, _):
                w16 = w_v[pl.ds(wbase + e16 * L, L)]
                for j in range(L):
                    wt = w16[j]
                    e = e16 * L + j
                    for g in range(COLS // L):
                        rows_v[e, pl.ds(g * L, L)] = rows_v[e, pl.ds(g * L, L)] * wt
                return 0
            lax.fori_loop(0, CHUNK // L, body, 0)

        def group(gidx, _):
            cbase = s * n_chunks + gidx * GRP
            ebase = (s * n_chunks + gidx * GRP) * CHUNK

            pltpu.sync_copy(src_hbm.at[pl.ds(cbase, GRP)], src_v)
            pltpu.sync_copy(dst_hbm.at[pl.ds(cbase, GRP)], dst_v)
            pltpu.sync_copy(w_hbm.at[pl.ds(ebase, GRP * CHUNK)], w_v)

            def gi(i, _):
                for g in range(CHUNK // L):
                    s16 = src_v[i, pl.ds(g * L, L)]
                    src_v[i, pl.ds(g * L, L)] = s16 * 2 + c
                return 0
            lax.fori_loop(0, GRP, gi, 0)

            for b in range(NBUF - 2):
                pltpu.async_copy(sup_hbm.at[src_v.at[b]], bufs[b], gsem[b])

            def rounds(jj, _):
                for b in range(NBUF):
                    i = jj * NBUF + b
                    r = (b - 2) % NBUF

                    @pl.when(i >= 2)
                    def _():
                        pltpu.make_async_copy(
                            bufs[r], acc.at[dst_v.at[0]], ssem[r]).wait()

                    @pl.when(i + NBUF - 2 < GRP)
                    def _():
                        pltpu.async_copy(
                            sup_hbm.at[src_v.at[i + NBUF - 2]], bufs[r], gsem[r])

                    pltpu.make_async_copy(
                        sup_hbm.at[src_v.at[i]], bufs[b], gsem[b]).wait()
                    scale(bufs[b], i * CHUNK)
                    pltpu.async_copy(bufs[b], acc.at[dst_v.at[i]], ssem[b], add=True)
                return 0
            lax.fori_loop(0, GRP // NBUF, rounds, 0)

            for i in (GRP - 2, GRP - 1):
                pltpu.make_async_copy(
                    bufs[i % NBUF], acc.at[dst_v.at[0]], ssem[i % NBUF]).wait()
            return 0
        lax.fori_loop(0, n_groups, group, 0)

        plsc.subcore_barrier()

        def outj(j, _):
            r0 = rbase + j * OB
            pltpu.sync_copy(acc.at[pl.ds(r0, OB)], bufs[0].at[pl.ds(0, OB)])
            pltpu.sync_copy(bufs[0].at[pl.ds(0, OB)], out_hbm.at[pl.ds(r0, OB), c])
            return 0
        lax.fori_loop(0, rows_per_tile // OB, outj, 0)

    return k(sup2, dst, src, w)


def kernel(input, edge_index, edge_weight, W, b):
    n_nodes, d_in = input.shape
    n_edges = edge_weight.shape[0]

    support = _linear_tc(input, W, b)
    sup2 = support.reshape(n_nodes * NC, COLS)

    grain = NS * CHUNK * 8
    ep = ((n_edges + grain - 1) // grain) * grain
    pad = ep - n_edges
    dst = jnp.pad(edge_index[0], (0, pad)).reshape(ep // CHUNK, CHUNK)
    src = jnp.pad(edge_index[1], (0, pad)).reshape(ep // CHUNK, CHUNK)
    w = jnp.pad(edge_weight, (0, pad))

    out3 = _spmm_sc(sup2, dst, src, w, n_nodes, ep // (NS * CHUNK))
    return out3.reshape(n_nodes, NC * COLS)


# speedup vs baseline: 3.1722x; 1.0066x over previous
import functools

import jax
import jax.numpy as jnp
from jax import lax
from jax.experimental import pallas as pl
from jax.experimental.pallas import tpu as pltpu
from jax.experimental.pallas import tpu_sc as plsc

NC = 2
NS = 16
L = 16

CHUNK = 64
NBUF = 4
GRP = 40
COLS = 128


def _linear_tc(x, W, b):
    N, K = x.shape
    DOUT = W.shape[0]
    BM = 400

    def mm(x_ref, w_ref, b_ref, o_ref):
        o_ref[...] = lax.dot_general(
            x_ref[...], w_ref[...],
            dimension_numbers=(((1,), (1,)), ((), ())),
            preferred_element_type=jnp.float32,
        ) + b_ref[...]

    return pl.pallas_call(
        mm,
        grid=(N // BM,),
        in_specs=[
            pl.BlockSpec((BM, K), lambda i: (i, 0)),
            pl.BlockSpec((DOUT, K), lambda i: (0, 0)),
            pl.BlockSpec((1, DOUT), lambda i: (0, 0)),
        ],
        out_specs=pl.BlockSpec((BM, DOUT), lambda i: (i, 0)),
        out_shape=jax.ShapeDtypeStruct((N, DOUT), jnp.float32),
    )(x, W, b[None, :])


def _spmm_sc(sup2, dst, src, w, n_nodes, n_chunks):
    rows_per_tile = n_nodes // NS
    edges_per_tile = n_chunks * CHUNK
    OB = 25
    n_groups = n_chunks // GRP

    mesh = plsc.VectorSubcoreMesh(core_axis_name="c", subcore_axis_name="s")

    row_bufs = [pltpu.VMEM((CHUNK, COLS), jnp.float32) for _ in range(NBUF)]
    gsems = [pltpu.SemaphoreType.DMA for _ in range(NBUF)]
    ssems = [pltpu.SemaphoreType.DMA for _ in range(NBUF)]

    @functools.partial(
        pl.kernel,
        out_type=jax.ShapeDtypeStruct((n_nodes, NC, COLS), jnp.float32),
        mesh=mesh,
        scratch_types=[
            pltpu.VMEM_SHARED((n_nodes, COLS), jnp.float32),
            pltpu.VMEM((GRP, CHUNK), jnp.int32),
            pltpu.VMEM((GRP, CHUNK), jnp.int32),
            pltpu.VMEM((GRP * CHUNK,), jnp.float32),
        ] + row_bufs + gsems + ssems,
    )
    def k(sup_hbm, dst_hbm, src_hbm, w_hbm, out_hbm,
          acc, src_v, dst_v, w_v, *bufs_and_sems):
        bufs = bufs_and_sems[:NBUF]
        gsem = bufs_and_sems[NBUF:2 * NBUF]
        ssem = bufs_and_sems[2 * NBUF:3 * NBUF]
        c = lax.axis_index("c")
        s = lax.axis_index("s")
        rbase = s * rows_per_tile

        def zrow(i, _):
            for g in range(COLS // L):
                bufs[0][i, pl.ds(g * L, L)] = jnp.zeros((L,), jnp.float32)
            return 0
        lax.fori_loop(0, OB, zrow, 0)
        def zacc(j, _):
            pltpu.sync_copy(bufs[0].at[pl.ds(0, OB)], acc.at[pl.ds(rbase + j * OB, OB)])
            return 0
        lax.fori_loop(0, rows_per_tile // OB, zacc, 0)

        plsc.subcore_barrier()

        def scale(rows_v, wbase):
            def body(e16, _):
                w16 = w_v[pl.ds(wbase + e16 * L, L)]
                for j in range(L):
                    wt = w16[j]
                    e = e16 * L + j
                    for g in range(COLS // L):
                        rows_v[e, pl.ds(g * L, L)] = rows_v[e, pl.ds(g * L, L)] * wt
                return 0
            lax.fori_loop(0, CHUNK // L, body, 0)

        def group(gidx, _):
            cbase = s * n_chunks + gidx * GRP
            ebase = (s * n_chunks + gidx * GRP) * CHUNK

            pltpu.sync_copy(src_hbm.at[pl.ds(cbase, GRP)], src_v)
            pltpu.sync_copy(dst_hbm.at[pl.ds(cbase, GRP)], dst_v)
            pltpu.sync_copy(w_hbm.at[pl.ds(ebase, GRP * CHUNK)], w_v)

            def gi(i, _):
                for g in range(CHUNK // L):
                    s16 = src_v[i, pl.ds(g * L, L)]
                    src_v[i, pl.ds(g * L, L)] = s16 * 2 + c
                return 0
            lax.fori_loop(0, GRP, gi, 0)

            for b in range(NBUF - 2):
                pltpu.async_copy(sup_hbm.at[src_v.at[b]], bufs[b], gsem[b])

            def rounds(jj, _):
                for b in range(NBUF):
                    i = jj * NBUF + b
                    r = (b - 2) % NBUF

                    @pl.when(i >= 2)
                    def _():
                        pltpu.make_async_copy(
                            bufs[r], acc.at[dst_v.at[0]], ssem[r]).wait()

                    @pl.when(i + NBUF - 2 < GRP)
                    def _():
                        pltpu.async_copy(
                            sup_hbm.at[src_v.at[i + NBUF - 2]], bufs[r], gsem[r])

                    pltpu.make_async_copy(
                        sup_hbm.at[src_v.at[i]], bufs[b], gsem[b]).wait()
                    scale(bufs[b], i * CHUNK)
                    pltpu.async_copy(bufs[b], acc.at[dst_v.at[i]], ssem[b], add=True)
                return 0
            lax.fori_loop(0, GRP // NBUF, rounds, 0)

            for i in (GRP - 2, GRP - 1):
                pltpu.make_async_copy(
                    bufs[i % NBUF], acc.at[dst_v.at[0]], ssem[i % NBUF]).wait()
            return 0
        lax.fori_loop(0, n_groups, group, 0)

        plsc.subcore_barrier()

        def outj(j, _):
            r0 = rbase + j * OB
            pltpu.sync_copy(acc.at[pl.ds(r0, OB)], bufs[0].at[pl.ds(0, OB)])
            pltpu.sync_copy(bufs[0].at[pl.ds(0, OB)], out_hbm.at[pl.ds(r0, OB), c])
            return 0
        lax.fori_loop(0, rows_per_tile // OB, outj, 0)

    return k(sup2, dst, src, w)


def kernel(input, edge_index, edge_weight, W, b):
    n_nodes, d_in = input.shape
    n_edges = edge_weight.shape[0]

    support = _linear_tc(input, W, b)
    sup2 = support.reshape(n_nodes * NC, COLS)

    grain = NS * CHUNK * 8
    ep = ((n_edges + grain - 1) // grain) * grain
    pad = ep - n_edges
    dst = jnp.pad(edge_index[0], (0, pad)).reshape(ep // CHUNK, CHUNK)
    src = jnp.pad(edge_index[1], (0, pad)).reshape(ep // CHUNK, CHUNK)
    w = jnp.pad(edge_weight, (0, pad))

    out3 = _spmm_sc(sup2, dst, src, w, n_nodes, ep // (NS * CHUNK))
    return out3.reshape(n_nodes, NC * COLS)


# speedup vs baseline: 3.2109x; 1.0122x over previous
import functools

import jax
import jax.numpy as jnp
from jax import lax
from jax.experimental import pallas as pl
from jax.experimental.pallas import tpu as pltpu
from jax.experimental.pallas import tpu_sc as plsc

NC = 2
NS = 16
L = 16

CHUNK = 64
NBUF = 4
GRP = 40
COLS = 128


def _linear_tc(x, W, b):
    N, K = x.shape
    DOUT = W.shape[0]
    BM = 400

    def mm(x_ref, w_ref, b_ref, o_ref):
        o_ref[...] = lax.dot_general(
            x_ref[...], w_ref[...],
            dimension_numbers=(((1,), (1,)), ((), ())),
            preferred_element_type=jnp.float32,
        ) + b_ref[...]

    return pl.pallas_call(
        mm,
        grid=(N // BM,),
        in_specs=[
            pl.BlockSpec((BM, K), lambda i: (i, 0)),
            pl.BlockSpec((DOUT, K), lambda i: (0, 0)),
            pl.BlockSpec((1, DOUT), lambda i: (0, 0)),
        ],
        out_specs=pl.BlockSpec((BM, DOUT), lambda i: (i, 0)),
        out_shape=jax.ShapeDtypeStruct((N, DOUT), jnp.float32),
    )(x, W, b[None, :])


def _spmm_sc(sup2, dst, src, w, n_nodes, n_chunks):
    rows_per_tile = n_nodes // NS
    edges_per_tile = n_chunks * CHUNK
    OB = 25
    n_groups = n_chunks // GRP

    mesh = plsc.VectorSubcoreMesh(core_axis_name="c", subcore_axis_name="s")

    row_bufs = [pltpu.VMEM((CHUNK, COLS), jnp.float32) for _ in range(NBUF)]
    gsems = [pltpu.SemaphoreType.DMA for _ in range(NBUF)]
    ssems = [pltpu.SemaphoreType.DMA for _ in range(NBUF)]

    @functools.partial(
        pl.kernel,
        out_type=jax.ShapeDtypeStruct((n_nodes, NC, COLS), jnp.float32),
        mesh=mesh,
        scratch_types=[
            pltpu.VMEM_SHARED((n_nodes, COLS), jnp.float32),
            pltpu.VMEM((GRP, CHUNK), jnp.int32),
            pltpu.VMEM((GRP, CHUNK), jnp.int32),
            pltpu.VMEM((GRP * CHUNK,), jnp.float32),
        ] + row_bufs + gsems + ssems,
    )
    def k(sup_hbm, dst_hbm, src_hbm, w_hbm, out_hbm,
          acc, src_v, dst_v, w_v, *bufs_and_sems):
        bufs = bufs_and_sems[:NBUF]
        gsem = bufs_and_sems[NBUF:2 * NBUF]
        ssem = bufs_and_sems[2 * NBUF:3 * NBUF]
        c = lax.axis_index("c")
        s = lax.axis_index("s")
        rbase = s * rows_per_tile

        def zrow(i, _):
            for g in range(COLS // L):
                bufs[0][i, pl.ds(g * L, L)] = jnp.zeros((L,), jnp.float32)
            return 0
        lax.fori_loop(0, OB, zrow, 0)
        def zacc(j, _):
            pltpu.async_copy(bufs[0].at[pl.ds(0, OB)],
                             acc.at[pl.ds(rbase + j * OB, OB)], gsem[0])
            return 0
        lax.fori_loop(0, rows_per_tile // OB, zacc, 0)
        def zdrain(j, _):
            pltpu.make_async_copy(bufs[0].at[pl.ds(0, OB)],
                                  acc.at[pl.ds(rbase, OB)], gsem[0]).wait()
            return 0
        lax.fori_loop(0, rows_per_tile // OB, zdrain, 0)

        plsc.subcore_barrier()

        def scale(rows_v, wbase):
            def body(e16, _):
                w16 = w_v[pl.ds(wbase + e16 * L, L)]
                for j in range(L):
                    wt = w16[j]
                    e = e16 * L + j
                    for g in range(COLS // L):
                        rows_v[e, pl.ds(g * L, L)] = rows_v[e, pl.ds(g * L, L)] * wt
                return 0
            lax.fori_loop(0, CHUNK // L, body, 0)

        def group(gidx, _):
            cbase = s * n_chunks + gidx * GRP
            ebase = (s * n_chunks + gidx * GRP) * CHUNK

            pltpu.sync_copy(src_hbm.at[pl.ds(cbase, GRP)], src_v)
            pltpu.sync_copy(dst_hbm.at[pl.ds(cbase, GRP)], dst_v)
            pltpu.sync_copy(w_hbm.at[pl.ds(ebase, GRP * CHUNK)], w_v)

            def gi(i, _):
                for g in range(CHUNK // L):
                    s16 = src_v[i, pl.ds(g * L, L)]
                    src_v[i, pl.ds(g * L, L)] = s16 * 2 + c
                return 0
            lax.fori_loop(0, GRP, gi, 0)

            for b in range(NBUF - 2):
                pltpu.async_copy(sup_hbm.at[src_v.at[b]], bufs[b], gsem[b])

            def rounds(jj, _):
                for b in range(NBUF):
                    i = jj * NBUF + b
                    r = (b - 2) % NBUF

                    @pl.when(i >= 2)
                    def _():
                        pltpu.make_async_copy(
                            bufs[r], acc.at[dst_v.at[0]], ssem[r]).wait()

                    @pl.when(i + NBUF - 2 < GRP)
                    def _():
                        pltpu.async_copy(
                            sup_hbm.at[src_v.at[i + NBUF - 2]], bufs[r], gsem[r])

                    pltpu.make_async_copy(
                        sup_hbm.at[src_v.at[i]], bufs[b], gsem[b]).wait()
                    scale(bufs[b], i * CHUNK)
                    pltpu.async_copy(bufs[b], acc.at[dst_v.at[i]], ssem[b], add=True)
                return 0
            lax.fori_loop(0, GRP // NBUF, rounds, 0)

            for i in (GRP - 2, GRP - 1):
                pltpu.make_async_copy(
                    bufs[i % NBUF], acc.at[dst_v.at[0]], ssem[i % NBUF]).wait()
            return 0
        lax.fori_loop(0, n_groups, group, 0)

        plsc.subcore_barrier()

        n_out = rows_per_tile // OB

        def outj(j, _):
            r0 = rbase + j * OB
            for bb in range(NBUF):
                @pl.when(j % NBUF == bb)
                def _():
                    @pl.when(j >= NBUF)
                    def _():
                        pltpu.make_async_copy(
                            bufs[bb].at[pl.ds(0, OB)],
                            out_hbm.at[pl.ds(rbase, OB), c], ssem[bb]).wait()
                    pltpu.sync_copy(acc.at[pl.ds(r0, OB)], bufs[bb].at[pl.ds(0, OB)])
                    pltpu.async_copy(bufs[bb].at[pl.ds(0, OB)],
                                     out_hbm.at[pl.ds(r0, OB), c], ssem[bb])
            return 0
        lax.fori_loop(0, n_out, outj, 0)
        for bb in range(min(NBUF, n_out)):
            pltpu.make_async_copy(
                bufs[bb].at[pl.ds(0, OB)],
                out_hbm.at[pl.ds(rbase, OB), c], ssem[bb]).wait()

    return k(sup2, dst, src, w)


def kernel(input, edge_index, edge_weight, W, b):
    n_nodes, d_in = input.shape
    n_edges = edge_weight.shape[0]

    support = _linear_tc(input, W, b)
    sup2 = support.reshape(n_nodes * NC, COLS)

    grain = NS * CHUNK * 8
    ep = ((n_edges + grain - 1) // grain) * grain
    pad = ep - n_edges
    dst = jnp.pad(edge_index[0], (0, pad)).reshape(ep // CHUNK, CHUNK)
    src = jnp.pad(edge_index[1], (0, pad)).reshape(ep // CHUNK, CHUNK)
    w = jnp.pad(edge_weight, (0, pad))

    out3 = _spmm_sc(sup2, dst, src, w, n_nodes, ep // (NS * CHUNK))
    return out3.reshape(n_nodes, NC * COLS)


# speedup vs baseline: 3.2296x; 1.0058x over previous
import functools

import jax
import jax.numpy as jnp
from jax import lax
from jax.experimental import pallas as pl
from jax.experimental.pallas import tpu as pltpu
from jax.experimental.pallas import tpu_sc as plsc

NC = 2
NS = 16
L = 16

CHUNK = 64
NBUF = 4
GRP = 40
COLS = 128


def _linear_tc(x, W, b):
    N, K = x.shape
    DOUT = W.shape[0]
    BM = 400

    def mm(x_ref, w_ref, b_ref, o_ref):
        o_ref[...] = lax.dot_general(
            x_ref[...], w_ref[...],
            dimension_numbers=(((1,), (1,)), ((), ())),
            preferred_element_type=jnp.float32,
        ) + b_ref[...]

    return pl.pallas_call(
        mm,
        grid=(N // BM,),
        in_specs=[
            pl.BlockSpec((BM, K), lambda i: (i, 0)),
            pl.BlockSpec((DOUT, K), lambda i: (0, 0)),
            pl.BlockSpec((1, DOUT), lambda i: (0, 0)),
        ],
        out_specs=pl.BlockSpec((BM, DOUT), lambda i: (i, 0)),
        out_shape=jax.ShapeDtypeStruct((N, DOUT), jnp.float32),
    )(x, W, b[None, :])


def _spmm_sc(sup2, dst, src, w, n_nodes, n_chunks):
    rows_per_tile = n_nodes // NS
    edges_per_tile = n_chunks * CHUNK
    OB = 25
    n_groups = n_chunks // GRP

    mesh = plsc.VectorSubcoreMesh(core_axis_name="c", subcore_axis_name="s")

    row_bufs = [pltpu.VMEM((CHUNK, COLS), jnp.float32) for _ in range(NBUF)]
    gsems = [pltpu.SemaphoreType.DMA for _ in range(NBUF)]
    ssems = [pltpu.SemaphoreType.DMA for _ in range(NBUF)]

    @functools.partial(
        pl.kernel,
        out_type=jax.ShapeDtypeStruct((n_nodes, NC, COLS), jnp.float32),
        mesh=mesh,
        scratch_types=[
            pltpu.VMEM_SHARED((n_nodes, COLS), jnp.float32),
            pltpu.VMEM((GRP, CHUNK), jnp.int32),
            pltpu.VMEM((GRP, CHUNK), jnp.int32),
            pltpu.VMEM((GRP * CHUNK,), jnp.float32),
        ] + row_bufs + gsems + ssems,
    )
    def k(sup_hbm, dst_hbm, src_hbm, w_hbm, out_hbm,
          acc, src_v, dst_v, w_v, *bufs_and_sems):
        bufs = bufs_and_sems[:NBUF]
        gsem = bufs_and_sems[NBUF:2 * NBUF]
        ssem = bufs_and_sems[2 * NBUF:3 * NBUF]
        c = lax.axis_index("c")
        s = lax.axis_index("s")
        rbase = s * rows_per_tile

        def zrow(i, _):
            for g in range(COLS // L):
                bufs[0][i, pl.ds(g * L, L)] = jnp.zeros((L,), jnp.float32)
            return 0
        lax.fori_loop(0, OB, zrow, 0)
        def zacc(j, _):
            pltpu.async_copy(bufs[0].at[pl.ds(0, OB)],
                             acc.at[pl.ds(rbase + j * OB, OB)], gsem[0])
            return 0
        lax.fori_loop(0, rows_per_tile // OB, zacc, 0)
        def zdrain(j, _):
            pltpu.make_async_copy(bufs[0].at[pl.ds(0, OB)],
                                  acc.at[pl.ds(rbase, OB)], gsem[0]).wait()
            return 0
        lax.fori_loop(0, rows_per_tile // OB, zdrain, 0)

        plsc.subcore_barrier()

        def scale(rows_v, wbase):
            def body(e16, _):
                w16 = w_v[pl.ds(wbase + e16 * L, L)]
                for j in range(L):
                    wt = w16[j]
                    e = e16 * L + j
                    for g in range(COLS // L):
                        rows_v[e, pl.ds(g * L, L)] = rows_v[e, pl.ds(g * L, L)] * wt
                return 0
            lax.fori_loop(0, CHUNK // L, body, 0)

        def group(gidx, _):
            cbase = s * n_chunks + gidx * GRP
            ebase = (s * n_chunks + gidx * GRP) * CHUNK

            pltpu.async_copy(src_hbm.at[pl.ds(cbase, GRP)], src_v, gsem[0])
            pltpu.async_copy(dst_hbm.at[pl.ds(cbase, GRP)], dst_v, gsem[1])
            pltpu.async_copy(w_hbm.at[pl.ds(ebase, GRP * CHUNK)], w_v, gsem[2])
            pltpu.make_async_copy(src_hbm.at[pl.ds(cbase, GRP)], src_v, gsem[0]).wait()
            pltpu.make_async_copy(dst_hbm.at[pl.ds(cbase, GRP)], dst_v, gsem[1]).wait()
            pltpu.make_async_copy(w_hbm.at[pl.ds(ebase, GRP * CHUNK)], w_v, gsem[2]).wait()

            def gi(i, _):
                for g in range(CHUNK // L):
                    s16 = src_v[i, pl.ds(g * L, L)]
                    src_v[i, pl.ds(g * L, L)] = s16 * 2 + c
                return 0
            lax.fori_loop(0, GRP, gi, 0)

            for b in range(NBUF - 2):
                pltpu.async_copy(sup_hbm.at[src_v.at[b]], bufs[b], gsem[b])

            def rounds(jj, _):
                for b in range(NBUF):
                    i = jj * NBUF + b
                    r = (b - 2) % NBUF

                    @pl.when(i >= 2)
                    def _():
                        pltpu.make_async_copy(
                            bufs[r], acc.at[dst_v.at[0]], ssem[r]).wait()

                    @pl.when(i + NBUF - 2 < GRP)
                    def _():
                        pltpu.async_copy(
                            sup_hbm.at[src_v.at[i + NBUF - 2]], bufs[r], gsem[r])

                    pltpu.make_async_copy(
                        sup_hbm.at[src_v.at[i]], bufs[b], gsem[b]).wait()
                    scale(bufs[b], i * CHUNK)
                    pltpu.async_copy(bufs[b], acc.at[dst_v.at[i]], ssem[b], add=True)
                return 0
            lax.fori_loop(0, GRP // NBUF, rounds, 0)

            for i in (GRP - 2, GRP - 1):
                pltpu.make_async_copy(
                    bufs[i % NBUF], acc.at[dst_v.at[0]], ssem[i % NBUF]).wait()
            return 0
        lax.fori_loop(0, n_groups, group, 0)

        plsc.subcore_barrier()

        n_out = rows_per_tile // OB

        def outj(j, _):
            r0 = rbase + j * OB
            for bb in range(NBUF):
                @pl.when(j % NBUF == bb)
                def _():
                    @pl.when(j >= NBUF)
                    def _():
                        pltpu.make_async_copy(
                            bufs[bb].at[pl.ds(0, OB)],
                            out_hbm.at[pl.ds(rbase, OB), c], ssem[bb]).wait()
                    pltpu.sync_copy(acc.at[pl.ds(r0, OB)], bufs[bb].at[pl.ds(0, OB)])
                    pltpu.async_copy(bufs[bb].at[pl.ds(0, OB)],
                                     out_hbm.at[pl.ds(r0, OB), c], ssem[bb])
            return 0
        lax.fori_loop(0, n_out, outj, 0)
        for bb in range(min(NBUF, n_out)):
            pltpu.make_async_copy(
                bufs[bb].at[pl.ds(0, OB)],
                out_hbm.at[pl.ds(rbase, OB), c], ssem[bb]).wait()

    return k(sup2, dst, src, w)


def kernel(input, edge_index, edge_weight, W, b):
    n_nodes, d_in = input.shape
    n_edges = edge_weight.shape[0]

    support = _linear_tc(input, W, b)
    sup2 = support.reshape(n_nodes * NC, COLS)

    grain = NS * CHUNK * 8
    ep = ((n_edges + grain - 1) // grain) * grain
    pad = ep - n_edges
    dst = jnp.pad(edge_index[0], (0, pad)).reshape(ep // CHUNK, CHUNK)
    src = jnp.pad(edge_index[1], (0, pad)).reshape(ep // CHUNK, CHUNK)
    w = jnp.pad(edge_weight, (0, pad))

    out3 = _spmm_sc(sup2, dst, src, w, n_nodes, ep // (NS * CHUNK))
    return out3.reshape(n_nodes, NC * COLS)
